# baseline (speedup 1.0000x reference)
import functools

import jax
import jax.numpy as jnp
from jax import lax
from jax.experimental import pallas as pl
from jax.experimental.pallas import tpu as pltpu
from jax.experimental.pallas import tpu_sc as plsc

R, C, H, B, E = 64, 200000, 16, 16, 50000
BE = B * E
BLK = 16384
GRID = (C + BLK - 1) // BLK
CPAD = GRID * BLK
NW = 32
CHUNK = BE // NW
TILE = 4096
GROUPS = TILE // 16
NB = 48

D1 = 3
D2OFF = 2
D2 = GRID - D2OFF
PS = 40960


def _dense_body(rc_ref, g_ref, gam_ref, bet_ref, w1a_ref, hrel_ref,
                w2_ref, b2_ref, s_ref):
    rc = rc_ref[...]
    mg = lax.dot_general(g_ref[...], rc, (((0,), (0,)), ((), ())),
                         preferred_element_type=jnp.float32)
    msg = mg[:H]
    r = 1.0 / (mg[H:H + 1] + 1e-6)
    x = msg * r
    mu = jnp.mean(x, axis=0, keepdims=True)
    var = jnp.mean((x - mu) * (x - mu), axis=0, keepdims=True)
    x = (x - mu) * lax.rsqrt(var + 1e-5) * gam_ref[...] + bet_ref[...]
    x = jnp.maximum(x, 0.0)
    h = lax.dot_general(w1a_ref[...], x, (((0,), (0,)), ((), ())),
                        preferred_element_type=jnp.float32)
    h = jnp.maximum(h + hrel_ref[...], 0.0)
    s = lax.dot_general(w2_ref[...], h, (((0,), (0,)), ((), ())),
                        preferred_element_type=jnp.float32)
    s_ref[...] = s + b2_ref[...]


def _dense_scores(rule_count, G, ln_gamma, ln_beta, W1a, h_rel, W2, b2,
                  nblk, off):
    return pl.pallas_call(
        _dense_body,
        grid=(nblk,),
        in_specs=[
            pl.BlockSpec((R, BLK), lambda i: (0, i + off)),
            pl.BlockSpec((R, H + 1), lambda i: (0, 0)),
            pl.BlockSpec((H, 1), lambda i: (0, 0)),
            pl.BlockSpec((H, 1), lambda i: (0, 0)),
            pl.BlockSpec((H, 128), lambda i: (0, 0)),
            pl.BlockSpec((128, 1), lambda i: (0, 0)),
            pl.BlockSpec((128, 1), lambda i: (0, 0)),
            pl.BlockSpec((1, 1), lambda i: (0, 0)),
        ],
        out_specs=pl.BlockSpec((1, BLK), lambda i: (0, i)),
        out_shape=jax.ShapeDtypeStruct((1, nblk * BLK), jnp.float32),
        compiler_params=pltpu.CompilerParams(
            dimension_semantics=("arbitrary",)),
    )(rule_count, G, ln_gamma, ln_beta, W1a, h_rel, W2, b2)


def _sc_scatter(cand_pad, s_half, init_buf, bounds,
                pos_lo, pos_hi, s_off, init_from_full):
    mesh = plsc.VectorSubcoreMesh(core_axis_name="c", subcore_axis_name="s")

    @functools.partial(
        pl.kernel,
        mesh=mesh,
        out_type=jax.ShapeDtypeStruct((BE,), jnp.float32),
        compiler_params=pltpu.CompilerParams(needs_layout_passes=False),
        scratch_types=[
            pltpu.VMEM((CHUNK,), jnp.float32),
            pltpu.VMEM((TILE + 16,), jnp.int32),
            pltpu.VMEM((TILE,), jnp.float32),
            pltpu.VMEM((NB,), jnp.int32),
            pltpu.SemaphoreType.DMA,
            pltpu.SemaphoreType.DMA,
            pltpu.SemaphoreType.DMA,
        ],
    )
    def k(cand_hbm, s_hbm, init_hbm, bounds_hbm, out_hbm,
          chunk, cbuf, sbuf, bv, csem, ssem, bsem):
        cid = lax.axis_index("c")
        sid = lax.axis_index("s")
        wid = cid * 16 + sid
        ostart = pl.multiple_of(wid * CHUNK, 8)
        oend = ostart + CHUNK
        if init_from_full:
            istart = ostart
        else:
            istart = pl.multiple_of(lax.rem(ostart, E), 8)
        init_cp = pltpu.async_copy(init_hbm.at[pl.ds(istart, CHUNK)], chunk,
                                   bsem)
        pltpu.sync_copy(bounds_hbm, bv)
        bw = bv[pl.ds(wid, 16)]
        lo = jnp.minimum(jnp.maximum(bw[0], pos_lo), pos_hi)
        hi = jnp.minimum(jnp.maximum(bw[1], pos_lo), pos_hi)
        lo8 = pl.multiple_of(lo - lax.rem(lo, 8), 8)
        ntiles = (hi - lo8 + (TILE - 1)) // TILE
        lanes = lax.iota(jnp.int32, 16)
        init_cp.wait()

        def tile_body(t, carry):
            base = pl.multiple_of(lo8 + t * TILE, 8)
            cc = pltpu.async_copy(cand_hbm.at[pl.ds(base, TILE + 16)], cbuf, csem)
            sc = pltpu.async_copy(s_hbm.at[pl.ds(base - s_off, TILE)], sbuf, ssem)
            cc.wait()
            sc.wait()
            for g in range(GROUPS):
                off = g * 16
                vc = cbuf[pl.ds(off, 16)]
                vn = cbuf[pl.ds(off + 1, 16)]
                vs = sbuf[pl.ds(off, 16)]
                idx = vc - ostart
                keep = (vc >= ostart) & (vc < oend) & (vc != vn)
                posv = lanes + (base + off)
                if init_from_full:
                    keep = keep & (posv >= pos_lo)
                else:
                    keep = keep & (posv < pos_hi)
                bv = plsc.load_gather(chunk, [idx], mask=keep)
                plsc.store_scatter(chunk, [idx], vs + bv, mask=keep)
            return carry

        lax.fori_loop(0, ntiles, tile_body, 0)
        pltpu.sync_copy(chunk, out_hbm.at[pl.ds(ostart, CHUNK)])

    return k(cand_pad, s_half, init_buf, bounds)


def kernel(rule_count, rule_emb, candidate_set, all_r, relation_table,
           ln_gamma, ln_beta, W1, b1, W2, b2, bias):
    rel = relation_table[all_r]
    G = jnp.concatenate([rule_emb, jnp.ones((R, 1), jnp.float32)], axis=1)
    h_rel = (rel @ W1[H:]) + b1
    cand_pad = jnp.concatenate(
        [candidate_set, jnp.full((CPAD - C,), BE, dtype=jnp.int32)])
    gam = ln_gamma.reshape(H, 1)
    bet = ln_beta.reshape(H, 1)
    hrel = h_rel.reshape(128, 1)
    b2r = b2.reshape(1, 1)
    edges = jnp.arange(NB, dtype=jnp.int32) * CHUNK
    bounds = jnp.searchsorted(candidate_set, edges, side="left",
                              method="compare_all").astype(jnp.int32)
    s1 = _dense_scores(rule_count, G, gam, bet, W1[:H], hrel, W2, b2r,
                       D1, 0).reshape(D1 * BLK)
    out1 = _sc_scatter(cand_pad, s1, bias, bounds, 0, PS, 0, False)
    s2 = _dense_scores(rule_count, G, gam, bet, W1[:H], hrel, W2, b2r,
                       D2, D2OFF).reshape(D2 * BLK)
    out2 = _sc_scatter(cand_pad, s2, out1, bounds, PS, C, D2OFF * BLK, True)
    return out2.reshape(B, E)


# speedup vs baseline: 1.1325x; 1.1325x over previous
import functools

import jax
import jax.numpy as jnp
from jax import lax
from jax.experimental import pallas as pl
from jax.experimental.pallas import tpu as pltpu
from jax.experimental.pallas import tpu_sc as plsc

R, C, H, B, E = 64, 200000, 16, 16, 50000
BE = B * E
BLK = 16384
GRID = (C + BLK - 1) // BLK
CPAD = GRID * BLK
NW = 32
CHUNK = BE // NW
TILE = 2048
GROUPS = TILE // 16
NB = 48


def _dense_body(rc_ref, g_ref, gam_ref, bet_ref, w1a_ref, hrel_ref,
                w2_ref, b2_ref, s_ref):
    rc = rc_ref[...]
    mg = lax.dot_general(g_ref[...], rc, (((0,), (0,)), ((), ())),
                         preferred_element_type=jnp.float32)
    msg = mg[:H]
    r = 1.0 / (mg[H:H + 1] + 1e-6)
    x = msg * r
    mu = jnp.mean(x, axis=0, keepdims=True)
    var = jnp.mean((x - mu) * (x - mu), axis=0, keepdims=True)
    x = (x - mu) * lax.rsqrt(var + 1e-5) * gam_ref[...] + bet_ref[...]
    x = jnp.maximum(x, 0.0)
    h = lax.dot_general(w1a_ref[...], x, (((0,), (0,)), ((), ())),
                        preferred_element_type=jnp.float32)
    h = jnp.maximum(h + hrel_ref[...], 0.0)
    s = lax.dot_general(w2_ref[...], h, (((0,), (0,)), ((), ())),
                        preferred_element_type=jnp.float32)
    s_ref[...] = s + b2_ref[...]


def _dense_scores(rule_count, G, ln_gamma, ln_beta, W1a, h_rel, W2, b2):
    return pl.pallas_call(
        _dense_body,
        grid=(GRID,),
        in_specs=[
            pl.BlockSpec((R, BLK), lambda i: (0, i)),
            pl.BlockSpec((R, H + 1), lambda i: (0, 0)),
            pl.BlockSpec((H, 1), lambda i: (0, 0)),
            pl.BlockSpec((H, 1), lambda i: (0, 0)),
            pl.BlockSpec((H, 128), lambda i: (0, 0)),
            pl.BlockSpec((128, 1), lambda i: (0, 0)),
            pl.BlockSpec((128, 1), lambda i: (0, 0)),
            pl.BlockSpec((1, 1), lambda i: (0, 0)),
        ],
        out_specs=pl.BlockSpec((1, BLK), lambda i: (0, i)),
        out_shape=jax.ShapeDtypeStruct((1, CPAD), jnp.float32),
        compiler_params=pltpu.CompilerParams(
            dimension_semantics=("arbitrary",)),
    )(rule_count, G, ln_gamma, ln_beta, W1a, h_rel, W2, b2)


def _sc_scatter(cand_pad, s_pad, bias, bounds):
    mesh = plsc.VectorSubcoreMesh(core_axis_name="c", subcore_axis_name="s")

    @functools.partial(
        pl.kernel,
        mesh=mesh,
        out_type=jax.ShapeDtypeStruct((BE,), jnp.float32),
        compiler_params=pltpu.CompilerParams(needs_layout_passes=False),
        scratch_types=[
            pltpu.VMEM((CHUNK,), jnp.float32),
            pltpu.VMEM((TILE + 16,), jnp.int32),
            pltpu.VMEM((TILE,), jnp.float32),
            pltpu.VMEM((NB,), jnp.int32),
            pltpu.SemaphoreType.DMA,
            pltpu.SemaphoreType.DMA,
            pltpu.SemaphoreType.DMA,
        ],
    )
    def k(cand_hbm, s_hbm, bias_hbm, bounds_hbm, out_hbm,
          chunk, cbuf, sbuf, bv, csem, ssem, bsem):
        cid = lax.axis_index("c")
        sid = lax.axis_index("s")
        wid = cid * 16 + sid
        ostart = pl.multiple_of(wid * CHUNK, 8)
        oend = ostart + CHUNK
        estart = pl.multiple_of(lax.rem(ostart, E), 8)
        bias_cp = pltpu.async_copy(bias_hbm.at[pl.ds(estart, CHUNK)], chunk,
                                   bsem)
        pltpu.sync_copy(bounds_hbm, bv)
        bw = bv[pl.ds(wid, 16)]
        lo = bw[0]
        hi = bw[1]
        lo8 = pl.multiple_of(lo - lax.rem(lo, 8), 8)
        ntiles = (hi - lo8 + (TILE - 1)) // TILE
        bias_cp.wait()

        def tile_body(t, carry):
            base = pl.multiple_of(lo8 + t * TILE, 8)
            cc = pltpu.async_copy(cand_hbm.at[pl.ds(base, TILE + 16)], cbuf, csem)
            sc = pltpu.async_copy(s_hbm.at[pl.ds(base, TILE)], sbuf, ssem)
            cc.wait()
            sc.wait()
            for g in range(GROUPS):
                off = g * 16
                vc = cbuf[pl.ds(off, 16)]
                vn = cbuf[pl.ds(off + 1, 16)]
                vs = sbuf[pl.ds(off, 16)]
                idx = vc - ostart
                keep = (vc >= ostart) & (vc < oend) & (vc != vn)
                bv = plsc.load_gather(chunk, [idx], mask=keep)
                plsc.store_scatter(chunk, [idx], vs + bv, mask=keep)
            return carry

        lax.fori_loop(0, ntiles, tile_body, 0)
        pltpu.sync_copy(chunk, out_hbm.at[pl.ds(ostart, CHUNK)])

    return k(cand_pad, s_pad, bias, bounds)


def kernel(rule_count, rule_emb, candidate_set, all_r, relation_table,
           ln_gamma, ln_beta, W1, b1, W2, b2, bias):
    rel = relation_table[all_r]
    G = jnp.concatenate([rule_emb, jnp.ones((R, 1), jnp.float32)], axis=1)
    h_rel = (rel @ W1[H:]) + b1
    cand_pad = jnp.concatenate(
        [candidate_set, jnp.full((CPAD - C,), BE, dtype=jnp.int32)])
    s2d = _dense_scores(rule_count, G,
                        ln_gamma.reshape(H, 1), ln_beta.reshape(H, 1),
                        W1[:H], h_rel.reshape(128, 1),
                        W2, b2.reshape(1, 1))
    s_pad = s2d.reshape(CPAD)
    edges = jnp.arange(NB, dtype=jnp.int32) * CHUNK
    bounds = jnp.searchsorted(candidate_set, edges, side="left",
                              method="compare_all").astype(jnp.int32)
    out_flat = _sc_scatter(cand_pad, s_pad, bias, bounds)
    return out_flat.reshape(B, E)


# speedup vs baseline: 1.1480x; 1.0137x over previous
import functools

import jax
import jax.numpy as jnp
from jax import lax
from jax.experimental import pallas as pl
from jax.experimental.pallas import tpu as pltpu
from jax.experimental.pallas import tpu_sc as plsc

R, C, H, B, E = 64, 200000, 16, 16, 50000
BE = B * E
BLK = 16384
GRID = (C + BLK - 1) // BLK
CPAD = GRID * BLK
NW = 32
CHUNK = BE // NW
TILE = 2048
GROUPS = TILE // 16
NB = 48


def _dense_body(rc_ref, g_ref, gb_ref, w1a_ref, hb_ref, w2_ref, s_ref):
    rc = rc_ref[...]
    mg = lax.dot_general(g_ref[...], rc, (((0,), (0,)), ((), ())),
                         preferred_element_type=jnp.float32)
    msg = mg[:H]
    r = 1.0 / (mg[H:H + 1] + 1e-6)
    x = msg * r
    mu = jnp.mean(x, axis=0, keepdims=True)
    var = jnp.mean((x - mu) * (x - mu), axis=0, keepdims=True)
    gb = gb_ref[...]
    x = (x - mu) * lax.rsqrt(var + 1e-5) * gb[:, 0:1] + gb[:, 1:2]
    x = jnp.maximum(x, 0.0)
    h = lax.dot_general(w1a_ref[...], x, (((0,), (0,)), ((), ())),
                        preferred_element_type=jnp.float32)
    hb = hb_ref[...]
    h = jnp.maximum(h + hb[:128], 0.0)
    s = lax.dot_general(w2_ref[...], h, (((0,), (0,)), ((), ())),
                        preferred_element_type=jnp.float32)
    s_ref[...] = s + hb[128:129]


def _dense_scores(rule_count, G, gb, W1a, hb, W2):
    return pl.pallas_call(
        _dense_body,
        grid=(GRID,),
        in_specs=[
            pl.BlockSpec((R, BLK), lambda i: (0, i)),
            pl.BlockSpec((R, H + 1), lambda i: (0, 0)),
            pl.BlockSpec((H, 2), lambda i: (0, 0)),
            pl.BlockSpec((H, 128), lambda i: (0, 0)),
            pl.BlockSpec((129, 1), lambda i: (0, 0)),
            pl.BlockSpec((128, 1), lambda i: (0, 0)),
        ],
        out_specs=pl.BlockSpec((1, BLK), lambda i: (0, i)),
        out_shape=jax.ShapeDtypeStruct((1, CPAD), jnp.float32),
        compiler_params=pltpu.CompilerParams(
            dimension_semantics=("arbitrary",)),
    )(rule_count, G, gb, W1a, hb, W2)


def _sc_scatter(cand_pad, s_pad, bias, bounds):
    mesh = plsc.VectorSubcoreMesh(core_axis_name="c", subcore_axis_name="s")

    @functools.partial(
        pl.kernel,
        mesh=mesh,
        out_type=jax.ShapeDtypeStruct((BE,), jnp.float32),
        compiler_params=pltpu.CompilerParams(needs_layout_passes=False),
        scratch_types=[
            pltpu.VMEM((CHUNK,), jnp.float32),
            pltpu.VMEM((TILE + 16,), jnp.int32),
            pltpu.VMEM((TILE,), jnp.float32),
            pltpu.VMEM((NB,), jnp.int32),
            pltpu.SemaphoreType.DMA,
            pltpu.SemaphoreType.DMA,
            pltpu.SemaphoreType.DMA,
        ],
    )
    def k(cand_hbm, s_hbm, bias_hbm, bounds_hbm, out_hbm,
          chunk, cbuf, sbuf, bv, csem, ssem, bsem):
        cid = lax.axis_index("c")
        sid = lax.axis_index("s")
        wid = cid * 16 + sid
        ostart = pl.multiple_of(wid * CHUNK, 8)
        oend = ostart + CHUNK
        estart = pl.multiple_of(lax.rem(ostart, E), 8)
        bias_cp = pltpu.async_copy(bias_hbm.at[pl.ds(estart, CHUNK)], chunk,
                                   bsem)
        pltpu.sync_copy(bounds_hbm, bv)
        bw = bv[pl.ds(wid, 16)]
        lo = bw[0]
        hi = bw[1]
        lo8 = pl.multiple_of(lo - lax.rem(lo, 8), 8)
        ntiles = (hi - lo8 + (TILE - 1)) // TILE
        bias_cp.wait()

        def tile_body(t, carry):
            base = pl.multiple_of(lo8 + t * TILE, 8)
            cc = pltpu.async_copy(cand_hbm.at[pl.ds(base, TILE + 16)], cbuf, csem)
            sc = pltpu.async_copy(s_hbm.at[pl.ds(base, TILE)], sbuf, ssem)
            cc.wait()
            sc.wait()
            for g in range(GROUPS):
                off = g * 16
                vc = cbuf[pl.ds(off, 16)]
                vn = cbuf[pl.ds(off + 1, 16)]
                vs = sbuf[pl.ds(off, 16)]
                idx = vc - ostart
                keep = (vc >= ostart) & (vc < oend) & (vc != vn)
                bv = plsc.load_gather(chunk, [idx], mask=keep)
                plsc.store_scatter(chunk, [idx], vs + bv, mask=keep)
            return carry

        lax.fori_loop(0, ntiles, tile_body, 0)
        pltpu.sync_copy(chunk, out_hbm.at[pl.ds(ostart, CHUNK)])

    return k(cand_pad, s_pad, bias, bounds)


def kernel(rule_count, rule_emb, candidate_set, all_r, relation_table,
           ln_gamma, ln_beta, W1, b1, W2, b2, bias):
    rel = relation_table[all_r]
    G = jnp.concatenate([rule_emb, jnp.ones((R, 1), jnp.float32)], axis=1)
    h_rel = (rel @ W1[H:]) + b1
    gb = jnp.stack([ln_gamma, ln_beta], axis=1)
    hb = jnp.concatenate([h_rel, b2]).reshape(129, 1)
    cand_pad = jnp.concatenate(
        [candidate_set, jnp.full((CPAD - C,), BE, dtype=jnp.int32)])
    s2d = _dense_scores(rule_count, G, gb, W1[:H], hb, W2)
    s_pad = s2d.reshape(CPAD)
    edges = jnp.arange(NB, dtype=jnp.int32) * CHUNK
    bounds = jnp.searchsorted(candidate_set, edges, side="left",
                              method="compare_all").astype(jnp.int32)
    out_flat = _sc_scatter(cand_pad, s_pad, bias, bounds)
    return out_flat.reshape(B, E)
